# baseline (speedup 1.0000x reference)
import jax
import jax.numpy as jnp
from jax import lax
from jax.experimental import pallas as pl
from jax.experimental.pallas import tpu as pltpu
from jax.experimental.pallas import tpu_sc as plsc

_N_TOKENS = 32768
_D = 1024
_E = 8
_K = 2
_L = 16
_NC = 2
_NS = 16
_NW = _NC * _NS
_TPW = _N_TOKENS // _NW

_BT = 2048


def _gate_body(x_ref, w_ref, b_ref, out_ref):
    acc = jnp.dot(x_ref[...], w_ref[...], preferred_element_type=jnp.float32)
    out_ref[...] = (acc + b_ref[...]).T


_NT0 = 28672
_NT1 = _N_TOKENS - _NT0


def _gate_logits_chunk(x, W, b, c0_blocks, nb):
    return pl.pallas_call(
        _gate_body,
        grid=(nb,),
        in_specs=[
            pl.BlockSpec((_BT, _D), lambda i, c0=c0_blocks: (c0 + i, 0)),
            pl.BlockSpec((_D, _E), lambda i: (0, 0)),
            pl.BlockSpec((1, _E), lambda i: (0, 0)),
        ],
        out_specs=pl.BlockSpec((_E, _BT), lambda i: (0, i)),
        out_shape=jax.ShapeDtypeStruct((_E, nb * _BT), jnp.float32),
    )(x, W, b.reshape(1, _E))


def _make_topk(nt):
    tpw = nt // _NW

    def _topk_body(g_hbm, p1_hbm, p2_hbm, i1_hbm, i2_hbm,
                   g_v, p1_v, p2_v, i1_v, i2_v, sem):
        wid = lax.axis_index("s") * _NC + lax.axis_index("c")
        base = wid * tpw
        pltpu.async_copy(g_hbm.at[:, pl.ds(base, tpw)], g_v, sem).wait()

        def step(t, carry):
            off = t * _L
            m1 = g_v[0, pl.ds(off, _L)]
            i1 = jnp.zeros((_L,), jnp.int32)
            m2 = jnp.full((_L,), -jnp.inf, jnp.float32)
            i2 = i1
            for e in range(1, _E):
                ev = jnp.full((_L,), e, jnp.int32)
                v = g_v[e, pl.ds(off, _L)]
                gt1 = v > m1
                gt2 = v > m2
                m2 = jnp.where(gt1, m1, jnp.where(gt2, v, m2))
                i2 = jnp.where(gt1, i1, jnp.where(gt2, ev, i2))
                m1 = jnp.where(gt1, v, m1)
                i1 = jnp.where(gt1, ev, i1)
            d = jnp.exp(m2 - m1)
            p1 = 1.0 / (1.0 + d)
            p1_v[pl.ds(off, _L)] = p1
            p2_v[pl.ds(off, _L)] = 1.0 - p1
            i1_v[pl.ds(off, _L)] = i1
            i2_v[pl.ds(off, _L)] = i2
            return carry

        lax.fori_loop(0, tpw // _L, step, 0)
        cs = [
            pltpu.async_copy(p1_v, p1_hbm.at[pl.ds(base, tpw)], sem),
            pltpu.async_copy(p2_v, p2_hbm.at[pl.ds(base, tpw)], sem),
            pltpu.async_copy(i1_v, i1_hbm.at[pl.ds(base, tpw)], sem),
            pltpu.async_copy(i2_v, i2_hbm.at[pl.ds(base, tpw)], sem),
        ]
        for c in cs:
            c.wait()

    return pl.kernel(
        _topk_body,
        out_type=tuple(
            jax.ShapeDtypeStruct((nt,), dt)
            for dt in (jnp.float32, jnp.float32, jnp.int32, jnp.int32)),
        mesh=plsc.VectorSubcoreMesh(
            core_axis_name="c", subcore_axis_name="s",
            num_cores=_NC, num_subcores=_NS,
        ),
        scratch_types=[
            pltpu.VMEM((_E, tpw), jnp.float32),
            pltpu.VMEM((tpw,), jnp.float32),
            pltpu.VMEM((tpw,), jnp.float32),
            pltpu.VMEM((tpw,), jnp.int32),
            pltpu.VMEM((tpw,), jnp.int32),
            pltpu.SemaphoreType.DMA,
        ],
    )


_topk0 = _make_topk(_NT0)
_topk1 = _make_topk(_NT1)


def kernel(x, W, b):
    g0 = _gate_logits_chunk(x, W, b, 0, _NT0 // _BT)
    o0 = _topk0(g0)
    g1 = _gate_logits_chunk(x, W, b, _NT0 // _BT, _NT1 // _BT)
    o1 = _topk1(g1)
    p1, p2, i1, i2 = (jnp.concatenate([a, bb]) for a, bb in zip(o0, o1))
    return jnp.stack([p1, p2], axis=1), jnp.stack([i1, i2], axis=1)


# speedup vs baseline: 1.0665x; 1.0665x over previous
import jax
import jax.numpy as jnp
from jax import lax
from jax.experimental import pallas as pl
from jax.experimental.pallas import tpu as pltpu
from jax.experimental.pallas import tpu_sc as plsc

_N_TOKENS = 32768
_D = 1024
_E = 8
_K = 2
_L = 16
_NC = 2
_NS = 16
_NW = _NC * _NS
_TPW = _N_TOKENS // _NW

_BT = 2048


def _gate_body(x_ref, w_ref, b_ref, out_ref):
    acc = jnp.dot(x_ref[...], w_ref[...], preferred_element_type=jnp.float32)
    out_ref[...] = (acc + b_ref[...]).T


def _gate_logits_t(x, W, b):
    return pl.pallas_call(
        _gate_body,
        grid=(_N_TOKENS // _BT,),
        in_specs=[
            pl.BlockSpec((_BT, _D), lambda i: (i, 0)),
            pl.BlockSpec((_D, _E), lambda i: (0, 0)),
            pl.BlockSpec((1, _E), lambda i: (0, 0)),
        ],
        out_specs=pl.BlockSpec((_E, _BT), lambda i: (0, i)),
        out_shape=jax.ShapeDtypeStruct((_E, _N_TOKENS), jnp.float32),
    )(x, W, b.reshape(1, _E))


def _topk_body(g_hbm, p1_hbm, p2_hbm, i1_hbm, i2_hbm,
               g_v, p1_v, p2_v, i1_v, i2_v, sem):
    wid = lax.axis_index("s") * _NC + lax.axis_index("c")
    base = wid * _TPW
    pltpu.async_copy(g_hbm.at[:, pl.ds(base, _TPW)], g_v, sem).wait()

    def step(t, carry):
        off = t * _L
        m1 = g_v[0, pl.ds(off, _L)]
        i1 = jnp.zeros((_L,), jnp.int32)
        m2 = jnp.full((_L,), -jnp.inf, jnp.float32)
        i2 = i1
        for e in range(1, _E):
            ev = jnp.full((_L,), e, jnp.int32)
            v = g_v[e, pl.ds(off, _L)]
            gt1 = v > m1
            gt2 = v > m2
            m2 = jnp.where(gt1, m1, jnp.where(gt2, v, m2))
            i2 = jnp.where(gt1, i1, jnp.where(gt2, ev, i2))
            m1 = jnp.where(gt1, v, m1)
            i1 = jnp.where(gt1, ev, i1)
        d = jnp.exp(m2 - m1)
        p1 = 1.0 / (1.0 + d)
        p1_v[pl.ds(off, _L)] = p1
        p2_v[pl.ds(off, _L)] = 1.0 - p1
        i1_v[pl.ds(off, _L)] = i1
        i2_v[pl.ds(off, _L)] = i2
        return carry

    lax.fori_loop(0, _TPW // _L, step, 0)
    cs = [
        pltpu.async_copy(p1_v, p1_hbm.at[pl.ds(base, _TPW)], sem),
        pltpu.async_copy(p2_v, p2_hbm.at[pl.ds(base, _TPW)], sem),
        pltpu.async_copy(i1_v, i1_hbm.at[pl.ds(base, _TPW)], sem),
        pltpu.async_copy(i2_v, i2_hbm.at[pl.ds(base, _TPW)], sem),
    ]
    for c in cs:
        c.wait()


_topk = pl.kernel(
    _topk_body,
    out_type=tuple(
        jax.ShapeDtypeStruct((_N_TOKENS,), dt)
        for dt in (jnp.float32, jnp.float32, jnp.int32, jnp.int32)),
    mesh=plsc.VectorSubcoreMesh(
        core_axis_name="c", subcore_axis_name="s",
        num_cores=_NC, num_subcores=_NS,
    ),
    scratch_types=[
        pltpu.VMEM((_E, _TPW), jnp.float32),
        pltpu.VMEM((_TPW,), jnp.float32),
        pltpu.VMEM((_TPW,), jnp.float32),
        pltpu.VMEM((_TPW,), jnp.int32),
        pltpu.VMEM((_TPW,), jnp.int32),
        pltpu.SemaphoreType.DMA,
    ],
)


def kernel(x, W, b):
    gt = _gate_logits_t(x, W, b)
    p1, p2, i1, i2 = _topk(gt)
    return jnp.stack([p1, p2], axis=1), jnp.stack([i1, i2], axis=1)
